# baseline (speedup 1.0000x reference)
import jax
import jax.numpy as jnp
from jax import lax
from jax.experimental import pallas as pl
from jax.experimental.pallas import tpu as pltpu
from jax.experimental.pallas import tpu_sc as plsc

OBJ_GOAL = 8
OBJ_LAVA = 9
OBJ_AGENT = 10
COL_RED = 0
COL_GREEN = 1
COL_YELLOW = 4

_NBUF = 4


def _body(posj_ref, valj_ref, lay_ref, out_ref, scratch, sems):
    nbuf, bHW = scratch.shape[0], scratch.shape[1]
    blk = scratch.shape[1:]
    i = pl.program_id(0)
    n = pl.num_programs(0)
    s = lax.rem(i, nbuf)

    @pl.when(i >= nbuf)
    def _():
        pltpu.make_async_copy(scratch.at[s], out_ref.at[pl.ds(0, bHW)],
                              sems.at[s]).wait()

    hw_idx = lax.broadcasted_iota(jnp.int32, blk, 0) + i * bHW
    j_odd = lax.broadcasted_iota(jnp.int32, blk, 1) & 1
    posv = posj_ref[...]
    valv = valj_ref[...]
    layb = jnp.broadcast_to(lay_ref[...], blk)
    base = jnp.where(j_odd == 1, 0, layb)
    eq = (hw_idx == posv).astype(jnp.int32)
    scratch[s] = base + eq * (valv - base)
    pltpu.make_async_copy(scratch.at[s], out_ref.at[pl.ds(i * bHW, bHW)],
                          sems.at[s]).start()

    @pl.when(i == n - 1)
    def _():
        for k in range(nbuf):
            pltpu.make_async_copy(scratch.at[k], out_ref.at[pl.ds(0, bHW)],
                                  sems.at[k]).wait()


_B, _HW = 4096, 4096
_NW = 32
_LPW = _B // _NW
_WPR = _HW // 4
_CW = 128


def _sc_body(m_ref, g_ref, w_ref, mcols_v, gst_v, wst_v):
    wid = lax.axis_index("s") * 2 + lax.axis_index("c")
    lane0 = wid * _LPW
    zero = jnp.zeros((16,), jnp.int32)

    def chunk_body(ci, accs):
        pltpu.sync_copy(
            m_ref.at[pl.ds(ci * _CW, _CW), pl.ds(lane0, _LPW)], mcols_v)

        def row_body(r, a):
            w_idx = ci * _CW + r
            out = []
            for j in range(8):
                wv = mcols_v[r, pl.ds(j * 16, 16)]
                nz = wv != 0
                out.append(a[2 * j] + jnp.where(nz, w_idx, 0))
                out.append(a[2 * j + 1] + wv)
            return tuple(out)

        return lax.fori_loop(0, _CW, row_body, accs)

    accs = lax.fori_loop(0, _WPR // _CW, chunk_body, (zero,) * 16)
    for j in range(8):
        gst_v[pl.ds(j * 16, 16)] = accs[2 * j]
        wst_v[pl.ds(j * 16, 16)] = accs[2 * j + 1]
    pltpu.sync_copy(gst_v, g_ref.at[pl.ds(lane0, _LPW)])
    pltpu.sync_copy(wst_v, w_ref.at[pl.ds(lane0, _LPW)])


def _sc_scan(m32t):
    mesh = plsc.VectorSubcoreMesh(core_axis_name="c", subcore_axis_name="s")
    f = pl.kernel(
        _sc_body,
        mesh=mesh,
        out_type=[
            jax.ShapeDtypeStruct((_B,), jnp.int32),
            jax.ShapeDtypeStruct((_B,), jnp.int32),
        ],
        scratch_types=[
            pltpu.VMEM((_CW, _LPW), jnp.int32),
            pltpu.VMEM((_LPW,), jnp.int32),
            pltpu.VMEM((_LPW,), jnp.int32),
        ],
    )
    return f(m32t)


def kernel(layout, mask_agent):
    B = mask_agent.shape[0]
    H, W = layout.shape[1], layout.shape[2]
    HW = H * W
    NT = B // 128

    lay2d = layout.reshape(H, W).astype(jnp.int32)
    m8 = mask_agent.astype(jnp.bool_).astype(jnp.int8)
    m32 = jax.lax.bitcast_convert_type(m8.reshape(B, HW // 4, 4), jnp.int32)
    g, wval = _sc_scan(m32.T)
    boff = ((wval == 256).astype(jnp.int32)
            + (wval == 65536).astype(jnp.int32) * 2
            + (wval == 16777216).astype(jnp.int32) * 3)
    pos = g * 4 + boff
    lval = jnp.take(lay2d.reshape(HW), pos)
    color = jnp.where(lval == OBJ_LAVA, COL_YELLOW,
                      jnp.where(lval == OBJ_GOAL, COL_GREEN, COL_RED))

    j_odd = (jnp.arange(2 * NT, dtype=jnp.int32) & 1)[:, None]
    pos_t = pos.reshape(NT, 1, 128)
    posj = jnp.broadcast_to(pos_t, (NT, 2, 128)).reshape(1, 2 * NT, 128)
    col_t = color.reshape(NT, 1, 128)
    colj = jnp.broadcast_to(col_t, (NT, 2, 128)).reshape(2 * NT, 128)
    valj = jnp.where(j_odd == 1, colj, OBJ_AGENT).reshape(1, 2 * NT, 128)

    lay_r = jnp.broadcast_to(lay2d.reshape(HW, 1, 1), (HW, 1, 128))

    bHW = 128
    out5 = pl.pallas_call(
        _body,
        grid=(HW // bHW,),
        in_specs=[
            pl.BlockSpec((1, 2 * NT, 128), lambda i: (0, 0, 0)),
            pl.BlockSpec((1, 2 * NT, 128), lambda i: (0, 0, 0)),
            pl.BlockSpec((bHW, 1, 128), lambda i: (i, 0, 0)),
        ],
        out_specs=pl.BlockSpec(memory_space=pl.ANY),
        out_shape=jax.ShapeDtypeStruct((HW, 2 * NT, 128), jnp.int32),
        scratch_shapes=[
            pltpu.VMEM((_NBUF, bHW, 2 * NT, 128), jnp.int32),
            pltpu.SemaphoreType.DMA((_NBUF,)),
        ],
    )(posj, valj, lay_r)

    out = out5.reshape(H, W, NT, 2, 128).transpose(2, 4, 0, 1, 3)
    return out.reshape(B, H, W, 2)


# speedup vs baseline: 3.0938x; 3.0938x over previous
import jax
import jax.numpy as jnp
from jax import lax
from jax.experimental import pallas as pl
from jax.experimental.pallas import tpu as pltpu

OBJ_GOAL = 8
OBJ_LAVA = 9
OBJ_AGENT = 10
COL_RED = 0
COL_GREEN = 1
COL_YELLOW = 4

_NBUF = 4


def _body(posj_ref, valj_ref, lay_ref, out_ref, scratch, sems):
    nbuf, bHW = scratch.shape[0], scratch.shape[1]
    blk = scratch.shape[1:]
    i = pl.program_id(0)
    n = pl.num_programs(0)
    s = lax.rem(i, nbuf)

    @pl.when(i >= nbuf)
    def _():
        pltpu.make_async_copy(scratch.at[s], out_ref.at[pl.ds(0, bHW)],
                              sems.at[s]).wait()

    hw_idx = lax.broadcasted_iota(jnp.int32, blk, 0) + i * bHW
    j_odd = lax.broadcasted_iota(jnp.int32, blk, 1) & 1
    posv = posj_ref[...]
    valv = valj_ref[...]
    layb = jnp.broadcast_to(lay_ref[...], blk)
    base = jnp.where(j_odd == 1, 0, layb)
    eq = (hw_idx == posv).astype(jnp.int32)
    scratch[s] = base + eq * (valv - base)
    pltpu.make_async_copy(scratch.at[s], out_ref.at[pl.ds(i * bHW, bHW)],
                          sems.at[s]).start()

    @pl.when(i == n - 1)
    def _():
        for k in range(nbuf):
            pltpu.make_async_copy(scratch.at[k], out_ref.at[pl.ds(0, bHW)],
                                  sems.at[k]).wait()


def kernel(layout, mask_agent):
    B = mask_agent.shape[0]
    H, W = layout.shape[1], layout.shape[2]
    HW = H * W
    NT = B // 128

    lay2d = layout.reshape(H, W).astype(jnp.int32)
    m = mask_agent.astype(jnp.bool_)
    hwgrid = (jnp.arange(H, dtype=jnp.int32)[:, None] * W
              + jnp.arange(W, dtype=jnp.int32)[None, :])
    pos = jnp.sum(jnp.where(m, hwgrid[None], 0), axis=(1, 2))
    lval = jnp.sum(jnp.where(m, lay2d[None], 0), axis=(1, 2))
    color = jnp.where(lval == OBJ_LAVA, COL_YELLOW,
                      jnp.where(lval == OBJ_GOAL, COL_GREEN, COL_RED))

    j_odd = (jnp.arange(2 * NT, dtype=jnp.int32) & 1)[:, None]
    pos_t = pos.reshape(NT, 1, 128)
    posj = jnp.broadcast_to(pos_t, (NT, 2, 128)).reshape(1, 2 * NT, 128)
    col_t = color.reshape(NT, 1, 128)
    colj = jnp.broadcast_to(col_t, (NT, 2, 128)).reshape(2 * NT, 128)
    valj = jnp.where(j_odd == 1, colj, OBJ_AGENT).reshape(1, 2 * NT, 128)

    lay_r = jnp.broadcast_to(lay2d.reshape(HW, 1, 1), (HW, 1, 128))

    bHW = 128
    out5 = pl.pallas_call(
        _body,
        grid=(HW // bHW,),
        in_specs=[
            pl.BlockSpec((1, 2 * NT, 128), lambda i: (0, 0, 0)),
            pl.BlockSpec((1, 2 * NT, 128), lambda i: (0, 0, 0)),
            pl.BlockSpec((bHW, 1, 128), lambda i: (i, 0, 0)),
        ],
        out_specs=pl.BlockSpec(memory_space=pl.ANY),
        out_shape=jax.ShapeDtypeStruct((HW, 2 * NT, 128), jnp.int32),
        scratch_shapes=[
            pltpu.VMEM((_NBUF, bHW, 2 * NT, 128), jnp.int32),
            pltpu.SemaphoreType.DMA((_NBUF,)),
        ],
    )(posj, valj, lay_r)

    out = out5.reshape(H, W, NT, 2, 128).transpose(2, 4, 0, 1, 3)
    return out.reshape(B, H, W, 2)
